# baseline (speedup 1.0000x reference)
import functools

import jax
import jax.numpy as jnp
from jax import lax
from jax.experimental import pallas as pl
from jax.experimental.pallas import tpu as pltpu
from jax.experimental.pallas import tpu_sc as plsc

E = 320000
D = 128
ROW = 2 * D + 2
CW = 2 * D
NUM_TYPE = 22
NUM_STEREO = 6
NC = 2
NS = 16
NW = NC * NS
C = 128
U = 10
SUP = C * U
NSUPER = E // SUP
SJ = (NSUPER - 1) // NW + 1
V = 16


def _edge_embed_body(table, t_hbm, s_hbm, a_hbm, c_hbm, out_hbm,
                     t0, t1, s0, s1, a0, a1, c0, c1,
                     idx0, idx1, buf0, buf1,
                     lsem0, lsem1, gsem0, gsem1, wsem0, wsem1):
    wid = lax.axis_index("s") * NC + lax.axis_index("c")
    iota = lax.iota(jnp.int32, V)
    col_a = jnp.full((V,), CW, jnp.int32)
    col_c = jnp.full((V,), CW + 1, jnp.int32)
    ins = ((t0, s0, a0, c0, lsem0), (t1, s1, a1, c1, lsem1))
    idxs = (idx0, idx1)
    bufs = (buf0, buf1)
    gsems = (gsem0, gsem1)
    wsems = (wsem0, wsem1)

    def issue_loads(si, p):
        tv, sv, av, cv, lsem = ins[p]
        base = si * SUP
        pltpu.async_copy(t_hbm.at[pl.ds(base, SUP)], tv, lsem)
        pltpu.async_copy(s_hbm.at[pl.ds(base, SUP)], sv, lsem)
        pltpu.async_copy(a_hbm.at[pl.ds(base, SUP)], av, lsem)
        pltpu.async_copy(c_hbm.at[pl.ds(base, SUP)], cv, lsem)

    def drain_loads(p):
        tv, sv, av, cv, lsem = ins[p]
        pltpu.make_async_copy(t_hbm.at[pl.ds(0, SUP)], tv, lsem).wait()
        pltpu.make_async_copy(s_hbm.at[pl.ds(0, SUP)], sv, lsem).wait()
        pltpu.make_async_copy(a_hbm.at[pl.ds(0, SUP)], av, lsem).wait()
        pltpu.make_async_copy(c_hbm.at[pl.ds(0, SUP)], cv, lsem).wait()

    @pl.when(wid < NSUPER)
    def _():
        issue_loads(wid, 0)

    def super_step(j, jp):
        si = wid + j * NW
        valid_s = si < NSUPER
        tv, sv, av, cv, _ = ins[jp]

        @pl.when(valid_s)
        def _():
            drain_loads(jp)
            @pl.when(si + NW < NSUPER)
            def _():
                issue_loads(si + NW, 1 - jp)

        for u in range(U):
            b = u & 1
            idx_b, buf_b = idxs[b], bufs[b]

            @pl.when(valid_s)
            def _():
                if u < 2:
                    @pl.when(j >= 1)
                    def _():
                        pltpu.make_async_copy(
                            bufs[b], out_hbm.at[pl.ds(0, C)], wsems[b]).wait()
                else:
                    pltpu.make_async_copy(
                        bufs[b], out_hbm.at[pl.ds(0, C)], wsems[b]).wait()
                for i in range(C // V):
                    off = u * C + i * V
                    idx_b[pl.ds(i * V, V)] = (
                        tv[pl.ds(off, V)] * NUM_STEREO + sv[pl.ds(off, V)])
                pltpu.async_copy(
                    table.at[idx_b], buf_b.at[:, pl.ds(0, CW)], gsems[b])
                for i in range(C // V):
                    rows = iota + i * V
                    plsc.store_scatter(
                        buf_b, [rows, col_a], av[pl.ds(u * C + i * V, V)])
                    plsc.store_scatter(
                        buf_b, [rows, col_c], cv[pl.ds(u * C + i * V, V)])

            pb = 1 - b
            if u >= 1:
                prev_valid = valid_s
                base_prev = si * SUP + (u - 1) * C
            else:
                prev_valid = jnp.logical_and(j >= 1, si - NW < NSUPER)
                base_prev = (si - NW) * SUP + (U - 1) * C

            @pl.when(prev_valid)
            def _():
                pltpu.make_async_copy(
                    table.at[idxs[pb]], bufs[pb].at[:, pl.ds(0, CW)],
                    gsems[pb]).wait()
                pltpu.async_copy(
                    bufs[pb], out_hbm.at[pl.ds(base_prev, C)], wsems[pb])

    def outer(jj, carry):
        super_step(jj * 2, 0)
        super_step(jj * 2 + 1, 1)
        return carry

    lax.fori_loop(0, (SJ + 2) // 2, outer, 0)

    pltpu.make_async_copy(buf0, out_hbm.at[pl.ds(0, C)], wsem0).wait()
    pltpu.make_async_copy(buf1, out_hbm.at[pl.ds(0, C)], wsem1).wait()


_edge_embed = functools.partial(
    pl.kernel,
    out_type=jax.ShapeDtypeStruct((E, ROW), jnp.float32),
    mesh=plsc.VectorSubcoreMesh(core_axis_name="c", subcore_axis_name="s"),
    scratch_types=[
        pltpu.VMEM((SUP,), jnp.int32),
        pltpu.VMEM((SUP,), jnp.int32),
        pltpu.VMEM((SUP,), jnp.int32),
        pltpu.VMEM((SUP,), jnp.int32),
        pltpu.VMEM((SUP,), jnp.float32),
        pltpu.VMEM((SUP,), jnp.float32),
        pltpu.VMEM((SUP,), jnp.float32),
        pltpu.VMEM((SUP,), jnp.float32),
        pltpu.VMEM((C,), jnp.int32),
        pltpu.VMEM((C,), jnp.int32),
        pltpu.VMEM((C, ROW), jnp.float32),
        pltpu.VMEM((C, ROW), jnp.float32),
        pltpu.SemaphoreType.DMA,
        pltpu.SemaphoreType.DMA,
        pltpu.SemaphoreType.DMA,
        pltpu.SemaphoreType.DMA,
        pltpu.SemaphoreType.DMA,
        pltpu.SemaphoreType.DMA,
    ],
    compiler_params=pltpu.CompilerParams(needs_layout_passes=False),
)(_edge_embed_body)


@jax.jit
def kernel(type_, stereo, aromatic, conjugated, type_table, stereo_table):
    table = jnp.concatenate([
        jnp.broadcast_to(type_table[:, None, :], (NUM_TYPE, NUM_STEREO, D)),
        jnp.broadcast_to(stereo_table[None, :, :], (NUM_TYPE, NUM_STEREO, D)),
    ], axis=-1).reshape(NUM_TYPE * NUM_STEREO, CW)
    return _edge_embed(table, type_.astype(jnp.int32), stereo.astype(jnp.int32),
                       aromatic, conjugated)
